# baseline (speedup 1.0000x reference)
import functools

import jax
import jax.numpy as jnp
from jax import lax
from jax.experimental import pallas as pl
from jax.experimental.pallas import tpu as pltpu
from jax.experimental.pallas import tpu_sc as plsc

_B, _S, _D = 1, 2048, 1024
_E, _TOPK = 8, 2
_FF, _FF_SH = 256, 512
_T = _B * _S

_TILE = 256
_NT = 24
_ROWS = _NT * _TILE
_NC, _NS = 2, 16
_NW = _NC * _NS
_TPW = _T // _NW


def _silu(x):
    return x * (1.0 / (1.0 + jnp.exp(-x)))


def _router_body(flat_ref, rw_ref, logits_ref, w01_ref, pos_ref, te_ref):
    flat = flat_ref[...]
    logits = jnp.dot(flat, rw_ref[...], preferred_element_type=jnp.float32)
    logits_ref[...] = logits
    lmax = jnp.max(logits, axis=1, keepdims=True)
    p = jnp.exp(logits - lmax)
    lane = lax.broadcasted_iota(jnp.int32, (_T, _E), 1)
    m1 = jnp.max(p, axis=1, keepdims=True)
    i1 = jnp.min(jnp.where(p == m1, lane, _E), axis=1, keepdims=True)
    p2 = jnp.where(lane == i1, -jnp.inf, p)
    m2 = jnp.max(p2, axis=1, keepdims=True)
    i2 = jnp.min(jnp.where(p2 == m2, lane, _E), axis=1, keepdims=True)
    s = m1 + m2
    w01_ref[...] = jnp.concatenate([m1 / s, m2 / s], axis=1)

    oh1 = (lane == i1).astype(jnp.bfloat16)
    oh2 = (lane == i2).astype(jnp.bfloat16)
    cnt = oh1 + oh2
    r_i = lax.broadcasted_iota(jnp.int32, (_T, _T), 0)
    c_i = lax.broadcasted_iota(jnp.int32, (_T, _T), 1)
    ltri = (c_i < r_i).astype(jnp.bfloat16)
    x_excl = jnp.dot(ltri, cnt, preferred_element_type=jnp.float32)

    c_tot = x_excl[_T - 1:_T, :] + cnt[_T - 1:_T, :].astype(jnp.float32)
    tiles = ((c_tot + float(_TILE - 1)) * (1.0 / _TILE)).astype(jnp.int32)
    tiles = tiles.astype(jnp.float32)
    r8 = lax.broadcasted_iota(jnp.int32, (_E, _E), 0)
    c8 = lax.broadcasted_iota(jnp.int32, (_E, _E), 1)
    diag = jnp.where(r8 == c8, jnp.broadcast_to(tiles, (_E, _E)), 0.0)
    tiles_col = jnp.sum(diag, axis=1, keepdims=True)
    start = jnp.sum(jnp.where(r8 < c8, jnp.broadcast_to(tiles_col, (_E, _E)),
                              0.0), axis=0, keepdims=True)
    aligned = start * float(_TILE)

    rank1 = jnp.sum(jnp.where(lane == i1, x_excl, 0.0), axis=1, keepdims=True)
    rank2 = jnp.sum(jnp.where(lane == i2, x_excl, 0.0), axis=1, keepdims=True)
    off1 = jnp.sum(jnp.where(lane == i1, aligned, 0.0), axis=1, keepdims=True)
    off2 = jnp.sum(jnp.where(lane == i2, aligned, 0.0), axis=1, keepdims=True)
    pos_ref[...] = jnp.concatenate(
        [rank1 + off1, rank2 + off2], axis=1).astype(jnp.int32)

    start_col = jnp.sum(jnp.where(c8 < r8, jnp.broadcast_to(tiles, (_E, _E)),
                                  0.0), axis=1, keepdims=True)
    incl_col = start_col + tiles_col
    jt = lax.broadcasted_iota(jnp.int32, (_E, _NT), 1).astype(jnp.float32)
    te = jnp.sum((jnp.broadcast_to(incl_col, (_E, _NT)) <= jt)
                 .astype(jnp.float32), axis=0, keepdims=True)
    te = jnp.minimum(te, float(_E - 1))
    n_active = jnp.sum(tiles, axis=1, keepdims=True)
    te_ref[...] = jnp.concatenate([te, n_active], axis=1).astype(jnp.int32)


def _router_tc(flat, router_w, interpret=False):
    return pl.pallas_call(
        _router_body,
        out_shape=[
            jax.ShapeDtypeStruct((_T, _E), jnp.float32),
            jax.ShapeDtypeStruct((_T, 2), jnp.float32),
            jax.ShapeDtypeStruct((_T, 2), jnp.int32),
            jax.ShapeDtypeStruct((1, _NT + 1), jnp.int32),
        ],
        interpret=interpret,
    )(flat, router_w)


def _shared_body(flat_ref, g_ref, u_ref, d_ref, out_ref):
    flat = flat_ref[...]
    g = jnp.dot(flat, g_ref[...], preferred_element_type=jnp.float32)
    u = jnp.dot(flat, u_ref[...], preferred_element_type=jnp.float32)
    out_ref[...] = jnp.dot((_silu(g) * u).astype(jnp.bfloat16), d_ref[...],
                           preferred_element_type=jnp.float32)


def _shared_tc(flat, sh_gate, sh_up, sh_down, interpret=False):
    return pl.pallas_call(
        _shared_body,
        out_shape=jax.ShapeDtypeStruct((_T, _D), jnp.float32),
        interpret=interpret,
    )(flat, sh_gate, sh_up, sh_down)


def _grouped_body(te_ref, x_ref, wg_ref, wu_ref, wd_ref, y_ref):
    @pl.when(pl.program_id(0) < te_ref[_NT])
    def _():
        x = x_ref[...].astype(jnp.bfloat16)
        g = jnp.dot(x, wg_ref[0], preferred_element_type=jnp.float32)
        u = jnp.dot(x, wu_ref[0], preferred_element_type=jnp.float32)
        y_ref[...] = jnp.dot((_silu(g) * u).astype(jnp.bfloat16), wd_ref[0],
                             preferred_element_type=jnp.float32)


def _grouped_tc(tile_expert, x_sorted, w_gate, w_up, w_down, interpret=False):
    grid_spec = pltpu.PrefetchScalarGridSpec(
        num_scalar_prefetch=1,
        grid=(_NT,),
        in_specs=[
            pl.BlockSpec((_TILE, _D), lambda i, te: (i, 0)),
            pl.BlockSpec((1, _D, _FF), lambda i, te: (te[i], 0, 0)),
            pl.BlockSpec((1, _D, _FF), lambda i, te: (te[i], 0, 0)),
            pl.BlockSpec((1, _FF, _D), lambda i, te: (te[i], 0, 0)),
        ],
        out_specs=pl.BlockSpec((_TILE, _D), lambda i, te: (i, 0)),
    )
    return pl.pallas_call(
        _grouped_body,
        grid_spec=grid_spec,
        out_shape=jax.ShapeDtypeStruct((_ROWS, _D), jnp.float32),
        compiler_params=pltpu.CompilerParams(
            dimension_semantics=("arbitrary",)),
        interpret=interpret,
    )(tile_expert, x_sorted, w_gate, w_up, w_down)


def _dispatch_sc(flat, pos3):
    mesh = plsc.VectorSubcoreMesh(core_axis_name="c", subcore_axis_name="s")

    @functools.partial(
        pl.kernel, mesh=mesh,
        out_type=jax.ShapeDtypeStruct((_ROWS, _D), jnp.float32),
        scratch_types=[
            pltpu.VMEM((2, _TPW), jnp.int32),
            pltpu.VMEM((_TPW, _D), jnp.float32),
            pltpu.SemaphoreType.DMA,
        ],
    )
    def k(flat_hbm, pos_hbm, out_hbm, idx_v, rows_v, sem):
        wid = lax.axis_index("s") * _NC + lax.axis_index("c")
        base = wid * _TPW
        pltpu.sync_copy(pos_hbm.at[wid], idx_v)
        pltpu.sync_copy(flat_hbm.at[pl.ds(base, _TPW)], rows_v)
        c0 = pltpu.async_copy(rows_v, out_hbm.at[idx_v.at[0]], sem)
        c1 = pltpu.async_copy(rows_v, out_hbm.at[idx_v.at[1]], sem)
        c0.wait()
        c1.wait()

    return k(flat, pos3)


def _gather_sc(y, pos3):
    mesh = plsc.VectorSubcoreMesh(core_axis_name="c", subcore_axis_name="s")

    half = _TPW // 2

    @functools.partial(
        pl.kernel, mesh=mesh,
        out_type=[jax.ShapeDtypeStruct((_T, _D), jnp.float32),
                  jax.ShapeDtypeStruct((_T, _D), jnp.float32)],
        scratch_types=[
            pltpu.VMEM((4, half), jnp.int32),
            pltpu.VMEM((half, _D), jnp.float32),
            pltpu.VMEM((half, _D), jnp.float32),
            pltpu.SemaphoreType.DMA,
            pltpu.SemaphoreType.DMA,
        ],
    )
    def k(y_hbm, pos_hbm, y0_hbm, y1_hbm, idx_v, r0_v, r1_v, semg, sems):
        wid = lax.axis_index("s") * _NC + lax.axis_index("c")
        base = wid * _TPW
        pltpu.sync_copy(pos_hbm.at[wid], idx_v)
        for h in range(2):
            g0 = pltpu.async_copy(y_hbm.at[idx_v.at[h]], r0_v, semg)
            g1 = pltpu.async_copy(y_hbm.at[idx_v.at[2 + h]], r1_v, semg)
            g0.wait()
            s0 = pltpu.async_copy(
                r0_v, y0_hbm.at[pl.ds(base + h * half, half)], sems)
            g1.wait()
            s1 = pltpu.async_copy(
                r1_v, y1_hbm.at[pl.ds(base + h * half, half)], sems)
            s0.wait()
            s1.wait()

    return k(y, pos3)


def _combine_body(sh_ref, y0_ref, y1_ref, w01_ref, out_ref):
    w0 = w01_ref[:, 0:1]
    w1 = w01_ref[:, 1:2]
    out_ref[...] = sh_ref[...] + w0 * y0_ref[...] + w1 * y1_ref[...]


def _combine_tc(shared, y0, y1, w01, interpret=False):
    nblk = 4
    rows = _T // nblk
    return pl.pallas_call(
        _combine_body,
        grid=(nblk,),
        in_specs=[
            pl.BlockSpec((rows, _D), lambda i: (i, 0)),
            pl.BlockSpec((rows, _D), lambda i: (i, 0)),
            pl.BlockSpec((rows, _D), lambda i: (i, 0)),
            pl.BlockSpec((rows, 2), lambda i: (i, 0)),
        ],
        out_specs=pl.BlockSpec((rows, _D), lambda i: (i, 0)),
        out_shape=jax.ShapeDtypeStruct((_T, _D), jnp.float32),
        interpret=interpret,
    )(shared, y0, y1, w01)


@jax.jit
def kernel(hidden_states, router_w, w_gate, w_up, w_down,
           sh_gate, sh_up, sh_down):
    flat = hidden_states.reshape(_T, _D)
    flat_bf = flat.astype(jnp.bfloat16)
    logits, w01, pos01, te = _router_tc(flat, router_w)
    pos3 = pos01.reshape(_NW, _TPW, 2).transpose(0, 2, 1)
    pos4 = (pos01.reshape(_NW, 2, _TPW // 2, 2)
            .transpose(0, 3, 1, 2).reshape(_NW, 4, _TPW // 2))
    tile_expert = te.reshape(_NT + 1)
    x_sorted = _dispatch_sc(flat, pos3)
    shared = _shared_tc(flat_bf, sh_gate.astype(jnp.bfloat16),
                        sh_up.astype(jnp.bfloat16),
                        sh_down.astype(jnp.bfloat16))
    y = _grouped_tc(tile_expert, x_sorted, w_gate.astype(jnp.bfloat16),
                    w_up.astype(jnp.bfloat16), w_down.astype(jnp.bfloat16))
    y0, y1 = _gather_sc(y, pos4)
    out = _combine_tc(shared, y0, y1, w01)
    return out.reshape(_B, _S, _D), logits


# speedup vs baseline: 1.1432x; 1.1432x over previous
import functools

import jax
import jax.numpy as jnp
from jax import lax
from jax.experimental import pallas as pl
from jax.experimental.pallas import tpu as pltpu
from jax.experimental.pallas import tpu_sc as plsc

_B, _S, _D = 1, 2048, 1024
_E, _TOPK = 8, 2
_FF, _FF_SH = 256, 512
_T = _B * _S

_TILE = 256
_NT = 24
_ROWS = _NT * _TILE
_NC, _NS = 2, 16
_NW = _NC * _NS
_TPW = _T // _NW


def _silu(x):
    return x * (1.0 / (1.0 + jnp.exp(-x)))


def _router_body(flat_ref, rw_ref, logits_ref, w01_ref, pos_ref, te_ref):
    flat = flat_ref[...]
    logits = jnp.dot(flat, rw_ref[...], preferred_element_type=jnp.float32)
    logits_ref[...] = logits
    lmax = jnp.max(logits, axis=1, keepdims=True)
    p = jnp.exp(logits - lmax)
    lane = lax.broadcasted_iota(jnp.int32, (_T, _E), 1)
    m1 = jnp.max(p, axis=1, keepdims=True)
    i1 = jnp.min(jnp.where(p == m1, lane, _E), axis=1, keepdims=True)
    p2 = jnp.where(lane == i1, -jnp.inf, p)
    m2 = jnp.max(p2, axis=1, keepdims=True)
    i2 = jnp.min(jnp.where(p2 == m2, lane, _E), axis=1, keepdims=True)
    s = m1 + m2
    w01_ref[...] = jnp.concatenate([m1 / s, m2 / s], axis=1)

    oh1 = (lane == i1).astype(jnp.bfloat16)
    oh2 = (lane == i2).astype(jnp.bfloat16)
    cnt = oh1 + oh2
    r_i = lax.broadcasted_iota(jnp.int32, (_T, _T), 0)
    c_i = lax.broadcasted_iota(jnp.int32, (_T, _T), 1)
    ltri = (c_i < r_i).astype(jnp.bfloat16)
    x_excl = jnp.dot(ltri, cnt, preferred_element_type=jnp.float32)

    c_tot = x_excl[_T - 1:_T, :] + cnt[_T - 1:_T, :].astype(jnp.float32)
    tiles = ((c_tot + float(_TILE - 1)) * (1.0 / _TILE)).astype(jnp.int32)
    tiles = tiles.astype(jnp.float32)
    r8 = lax.broadcasted_iota(jnp.int32, (_E, _E), 0)
    c8 = lax.broadcasted_iota(jnp.int32, (_E, _E), 1)
    diag = jnp.where(r8 == c8, jnp.broadcast_to(tiles, (_E, _E)), 0.0)
    tiles_col = jnp.sum(diag, axis=1, keepdims=True)
    start = jnp.sum(jnp.where(r8 < c8, jnp.broadcast_to(tiles_col, (_E, _E)),
                              0.0), axis=0, keepdims=True)
    aligned = start * float(_TILE)

    rank1 = jnp.sum(jnp.where(lane == i1, x_excl, 0.0), axis=1, keepdims=True)
    rank2 = jnp.sum(jnp.where(lane == i2, x_excl, 0.0), axis=1, keepdims=True)
    off1 = jnp.sum(jnp.where(lane == i1, aligned, 0.0), axis=1, keepdims=True)
    off2 = jnp.sum(jnp.where(lane == i2, aligned, 0.0), axis=1, keepdims=True)
    pos_ref[...] = jnp.concatenate(
        [rank1 + off1, rank2 + off2], axis=1).astype(jnp.int32)

    start_col = jnp.sum(jnp.where(c8 < r8, jnp.broadcast_to(tiles, (_E, _E)),
                                  0.0), axis=1, keepdims=True)
    incl_col = start_col + tiles_col
    jt = lax.broadcasted_iota(jnp.int32, (_E, _NT), 1).astype(jnp.float32)
    te = jnp.sum((jnp.broadcast_to(incl_col, (_E, _NT)) <= jt)
                 .astype(jnp.float32), axis=0, keepdims=True)
    te = jnp.minimum(te, float(_E - 1))
    n_active = jnp.sum(tiles, axis=1, keepdims=True)
    te_ref[...] = jnp.concatenate([te, n_active], axis=1).astype(jnp.int32)


def _router_tc(flat, router_w, interpret=False):
    return pl.pallas_call(
        _router_body,
        out_shape=[
            jax.ShapeDtypeStruct((_T, _E), jnp.float32),
            jax.ShapeDtypeStruct((_T, 2), jnp.float32),
            jax.ShapeDtypeStruct((_T, 2), jnp.int32),
            jax.ShapeDtypeStruct((1, _NT + 1), jnp.int32),
        ],
        interpret=interpret,
    )(flat, router_w)


def _shared_body(flat_ref, g_ref, u_ref, d_ref, out_ref):
    flat = flat_ref[...]
    g = jnp.dot(flat, g_ref[...], preferred_element_type=jnp.float32)
    u = jnp.dot(flat, u_ref[...], preferred_element_type=jnp.float32)
    out_ref[...] = jnp.dot(_silu(g) * u, d_ref[...],
                           preferred_element_type=jnp.float32)


def _shared_tc(flat, sh_gate, sh_up, sh_down, interpret=False):
    return pl.pallas_call(
        _shared_body,
        out_shape=jax.ShapeDtypeStruct((_T, _D), jnp.float32),
        interpret=interpret,
    )(flat, sh_gate, sh_up, sh_down)


def _grouped_body(te_ref, x_ref, wg_ref, wu_ref, wd_ref, y_ref):
    @pl.when(pl.program_id(0) < te_ref[_NT])
    def _():
        x = x_ref[...]
        g = jnp.dot(x, wg_ref[0], preferred_element_type=jnp.float32)
        u = jnp.dot(x, wu_ref[0], preferred_element_type=jnp.float32)
        y_ref[...] = jnp.dot(_silu(g) * u, wd_ref[0],
                             preferred_element_type=jnp.float32)


def _grouped_tc(tile_expert, x_sorted, w_gate, w_up, w_down, interpret=False):
    grid_spec = pltpu.PrefetchScalarGridSpec(
        num_scalar_prefetch=1,
        grid=(_NT,),
        in_specs=[
            pl.BlockSpec((_TILE, _D), lambda i, te: (i, 0)),
            pl.BlockSpec((1, _D, _FF), lambda i, te: (te[i], 0, 0)),
            pl.BlockSpec((1, _D, _FF), lambda i, te: (te[i], 0, 0)),
            pl.BlockSpec((1, _FF, _D), lambda i, te: (te[i], 0, 0)),
        ],
        out_specs=pl.BlockSpec((_TILE, _D), lambda i, te: (i, 0)),
    )
    return pl.pallas_call(
        _grouped_body,
        grid_spec=grid_spec,
        out_shape=jax.ShapeDtypeStruct((_ROWS, _D), jnp.float32),
        compiler_params=pltpu.CompilerParams(
            dimension_semantics=("arbitrary",)),
        interpret=interpret,
    )(tile_expert, x_sorted, w_gate, w_up, w_down)


def _dispatch_sc(flat, pos3):
    mesh = plsc.VectorSubcoreMesh(core_axis_name="c", subcore_axis_name="s")

    @functools.partial(
        pl.kernel, mesh=mesh,
        out_type=jax.ShapeDtypeStruct((_ROWS, _D), jnp.float32),
        scratch_types=[
            pltpu.VMEM((2, _TPW), jnp.int32),
            pltpu.VMEM((_TPW, _D), jnp.float32),
            pltpu.SemaphoreType.DMA,
        ],
    )
    def k(flat_hbm, pos_hbm, out_hbm, idx_v, rows_v, sem):
        wid = lax.axis_index("s") * _NC + lax.axis_index("c")
        base = wid * _TPW
        pltpu.sync_copy(pos_hbm.at[wid], idx_v)
        pltpu.sync_copy(flat_hbm.at[pl.ds(base, _TPW)], rows_v)
        c0 = pltpu.async_copy(rows_v, out_hbm.at[idx_v.at[0]], sem)
        c1 = pltpu.async_copy(rows_v, out_hbm.at[idx_v.at[1]], sem)
        c0.wait()
        c1.wait()

    return k(flat, pos3)


def _gather_sc(y, pos3):
    mesh = plsc.VectorSubcoreMesh(core_axis_name="c", subcore_axis_name="s")

    half = _TPW // 2

    @functools.partial(
        pl.kernel, mesh=mesh,
        out_type=[jax.ShapeDtypeStruct((_T, _D), jnp.float32),
                  jax.ShapeDtypeStruct((_T, _D), jnp.float32)],
        scratch_types=[
            pltpu.VMEM((4, half), jnp.int32),
            pltpu.VMEM((half, _D), jnp.float32),
            pltpu.VMEM((half, _D), jnp.float32),
            pltpu.SemaphoreType.DMA,
            pltpu.SemaphoreType.DMA,
        ],
    )
    def k(y_hbm, pos_hbm, y0_hbm, y1_hbm, idx_v, r0_v, r1_v, semg, sems):
        wid = lax.axis_index("s") * _NC + lax.axis_index("c")
        base = wid * _TPW
        pltpu.sync_copy(pos_hbm.at[wid], idx_v)
        for h in range(2):
            g0 = pltpu.async_copy(y_hbm.at[idx_v.at[h]], r0_v, semg)
            g1 = pltpu.async_copy(y_hbm.at[idx_v.at[2 + h]], r1_v, semg)
            g0.wait()
            s0 = pltpu.async_copy(
                r0_v, y0_hbm.at[pl.ds(base + h * half, half)], sems)
            g1.wait()
            s1 = pltpu.async_copy(
                r1_v, y1_hbm.at[pl.ds(base + h * half, half)], sems)
            s0.wait()
            s1.wait()

    return k(y, pos3)


def _combine_body(sh_ref, y0_ref, y1_ref, w01_ref, out_ref):
    w0 = w01_ref[:, 0:1]
    w1 = w01_ref[:, 1:2]
    out_ref[...] = sh_ref[...] + w0 * y0_ref[...] + w1 * y1_ref[...]


def _combine_tc(shared, y0, y1, w01, interpret=False):
    nblk = 4
    rows = _T // nblk
    return pl.pallas_call(
        _combine_body,
        grid=(nblk,),
        in_specs=[
            pl.BlockSpec((rows, _D), lambda i: (i, 0)),
            pl.BlockSpec((rows, _D), lambda i: (i, 0)),
            pl.BlockSpec((rows, _D), lambda i: (i, 0)),
            pl.BlockSpec((rows, 2), lambda i: (i, 0)),
        ],
        out_specs=pl.BlockSpec((rows, _D), lambda i: (i, 0)),
        out_shape=jax.ShapeDtypeStruct((_T, _D), jnp.float32),
        interpret=interpret,
    )(shared, y0, y1, w01)


@jax.jit
def kernel(hidden_states, router_w, w_gate, w_up, w_down,
           sh_gate, sh_up, sh_down):
    flat = hidden_states.reshape(_T, _D)
    logits, w01, pos01, te = _router_tc(flat, router_w)
    pos3 = pos01.reshape(_NW, _TPW, 2).transpose(0, 2, 1)
    pos4 = (pos01.reshape(_NW, 2, _TPW // 2, 2)
            .transpose(0, 3, 1, 2).reshape(_NW, 4, _TPW // 2))
    tile_expert = te.reshape(_NT + 1)
    x_sorted = _dispatch_sc(flat, pos3)
    shared = _shared_tc(flat, sh_gate, sh_up, sh_down)
    y = _grouped_tc(tile_expert, x_sorted, w_gate, w_up, w_down)
    y0, y1 = _gather_sc(y, pos4)
    out = _combine_tc(shared, y0, y1, w01)
    return out.reshape(_B, _S, _D), logits


# speedup vs baseline: 1.2116x; 1.0598x over previous
import functools

import jax
import jax.numpy as jnp
from jax import lax
from jax.experimental import pallas as pl
from jax.experimental.pallas import tpu as pltpu
from jax.experimental.pallas import tpu_sc as plsc

_B, _S, _D = 1, 2048, 1024
_E, _TOPK = 8, 2
_FF, _FF_SH = 256, 512
_T = _B * _S

_TILE = 256
_NT = 24
_ROWS = _NT * _TILE
_NC, _NS = 2, 16
_NW = _NC * _NS
_TPW = _T // _NW


def _silu(x):
    return x * (1.0 / (1.0 + jnp.exp(-x)))


def _router_body(flat_ref, rw_ref, logits_ref, w01_ref, pos0_ref, pos1_ref,
                 te_ref):
    flat = flat_ref[...]
    logits = jnp.dot(flat, rw_ref[...], preferred_element_type=jnp.float32)
    logits_ref[...] = logits
    lt = jnp.transpose(logits)
    lmax = jnp.max(lt, axis=0, keepdims=True)
    p = jnp.exp(lt - lmax)
    sub = lax.broadcasted_iota(jnp.int32, (_E, _T), 0)
    m1 = jnp.max(p, axis=0, keepdims=True)
    i1 = jnp.min(jnp.where(p == m1, sub, _E), axis=0, keepdims=True)
    p2 = jnp.where(sub == i1, -jnp.inf, p)
    m2 = jnp.max(p2, axis=0, keepdims=True)
    i2 = jnp.min(jnp.where(p2 == m2, sub, _E), axis=0, keepdims=True)
    s = m1 + m2
    w01_ref[...] = jnp.concatenate([m1 / s, m2 / s], axis=0)

    oh1 = (sub == i1).astype(jnp.float32)
    oh2 = (sub == i2).astype(jnp.float32)
    cnt = (oh1 + oh2).reshape(128, 128)
    r1 = lax.broadcasted_iota(jnp.int32, (128, 128), 0)
    c1 = lax.broadcasted_iota(jnp.int32, (128, 128), 1)
    ut = (r1 < c1).astype(jnp.bfloat16)
    local = jnp.dot(cnt.astype(jnp.bfloat16), ut,
                    preferred_element_type=jnp.float32)
    rowsum = jnp.sum(cnt, axis=1, keepdims=True)
    bdl = ((r1 // 16 == c1 // 16) & (c1 % 16 < r1 % 16)).astype(jnp.bfloat16)
    prefix = jnp.dot(bdl, rowsum.astype(jnp.bfloat16),
                     preferred_element_type=jnp.float32)
    x_t = (local + prefix).reshape(_E, _T)

    c_col = jnp.sum(cnt, axis=1, keepdims=True).reshape(_E, 16).sum(
        axis=1, keepdims=True)
    tiles_col = ((c_col + float(_TILE - 1)) * (1.0 / _TILE)
                 ).astype(jnp.int32).astype(jnp.float32)
    r8 = lax.broadcasted_iota(jnp.int32, (_E, _E), 0)
    c8 = lax.broadcasted_iota(jnp.int32, (_E, _E), 1)
    l8 = (c8 < r8).astype(jnp.bfloat16)
    start_col = jnp.dot(l8, tiles_col.astype(jnp.bfloat16),
                        preferred_element_type=jnp.float32)
    aligned_col = start_col * float(_TILE)

    al_b = jnp.broadcast_to(aligned_col, (_E, _T))
    rank1 = jnp.sum(jnp.where(sub == i1, x_t + al_b, 0.0), axis=0,
                    keepdims=True)
    rank2 = jnp.sum(jnp.where(sub == i2, x_t + al_b, 0.0), axis=0,
                    keepdims=True)
    pos0_ref[...] = rank1.astype(jnp.int32).reshape(_T)
    pos1_ref[...] = rank2.astype(jnp.int32).reshape(_T)

    incl_col = start_col + tiles_col
    jt = lax.broadcasted_iota(jnp.int32, (_E, _NT), 1).astype(jnp.float32)
    te = jnp.sum((jnp.broadcast_to(incl_col, (_E, _NT)) <= jt)
                 .astype(jnp.float32), axis=0, keepdims=True)
    te = jnp.minimum(te, float(_E - 1))
    n_active = jnp.sum(tiles_col, axis=0, keepdims=True)
    te_ref[...] = jnp.concatenate([te, n_active], axis=1).astype(jnp.int32)


def _router_tc(flat, router_w, interpret=False):
    return pl.pallas_call(
        _router_body,
        out_shape=[
            jax.ShapeDtypeStruct((_T, _E), jnp.float32),
            jax.ShapeDtypeStruct((2, _T), jnp.float32),
            jax.ShapeDtypeStruct((_T,), jnp.int32),
            jax.ShapeDtypeStruct((_T,), jnp.int32),
            jax.ShapeDtypeStruct((1, _NT + 1), jnp.int32),
        ],
        interpret=interpret,
    )(flat, router_w)


def _shared_body(flat_ref, g_ref, u_ref, d_ref, out_ref):
    flat = flat_ref[...]
    g = jnp.dot(flat, g_ref[...], preferred_element_type=jnp.float32)
    u = jnp.dot(flat, u_ref[...], preferred_element_type=jnp.float32)
    out_ref[...] = jnp.dot(_silu(g) * u, d_ref[...],
                           preferred_element_type=jnp.float32)


def _shared_tc(flat, sh_gate, sh_up, sh_down, interpret=False):
    return pl.pallas_call(
        _shared_body,
        out_shape=jax.ShapeDtypeStruct((_T, _D), jnp.float32),
        interpret=interpret,
    )(flat, sh_gate, sh_up, sh_down)


def _grouped_body(te_ref, x_ref, wg_ref, wu_ref, wd_ref, y_ref):
    @pl.when(pl.program_id(0) < te_ref[_NT])
    def _():
        x = x_ref[...]
        g = jnp.dot(x, wg_ref[0], preferred_element_type=jnp.float32)
        u = jnp.dot(x, wu_ref[0], preferred_element_type=jnp.float32)
        y_ref[...] = jnp.dot(_silu(g) * u, wd_ref[0],
                             preferred_element_type=jnp.float32)


def _grouped_tc(tile_expert, x_sorted, w_gate, w_up, w_down, interpret=False):
    grid_spec = pltpu.PrefetchScalarGridSpec(
        num_scalar_prefetch=1,
        grid=(_NT,),
        in_specs=[
            pl.BlockSpec((_TILE, _D), lambda i, te: (i, 0)),
            pl.BlockSpec((1, _D, _FF), lambda i, te: (te[i], 0, 0)),
            pl.BlockSpec((1, _D, _FF), lambda i, te: (te[i], 0, 0)),
            pl.BlockSpec((1, _FF, _D), lambda i, te: (te[i], 0, 0)),
        ],
        out_specs=pl.BlockSpec((_TILE, _D), lambda i, te: (i, 0)),
    )
    return pl.pallas_call(
        _grouped_body,
        grid_spec=grid_spec,
        out_shape=jax.ShapeDtypeStruct((_ROWS, _D), jnp.float32),
        compiler_params=pltpu.CompilerParams(
            dimension_semantics=("arbitrary",)),
        interpret=interpret,
    )(tile_expert, x_sorted, w_gate, w_up, w_down)


def _dispatch_sc(flat, pos0, pos1):
    mesh = plsc.VectorSubcoreMesh(core_axis_name="c", subcore_axis_name="s")

    @functools.partial(
        pl.kernel, mesh=mesh,
        out_type=jax.ShapeDtypeStruct((_ROWS, _D), jnp.float32),
        scratch_types=[
            pltpu.VMEM((_TPW,), jnp.int32),
            pltpu.VMEM((_TPW,), jnp.int32),
            pltpu.VMEM((_TPW, _D), jnp.float32),
            pltpu.SemaphoreType.DMA,
        ],
    )
    def k(flat_hbm, p0_hbm, p1_hbm, out_hbm, idx0_v, idx1_v, rows_v, sem):
        wid = lax.axis_index("s") * _NC + lax.axis_index("c")
        base = wid * _TPW
        pltpu.sync_copy(p0_hbm.at[pl.ds(base, _TPW)], idx0_v)
        pltpu.sync_copy(p1_hbm.at[pl.ds(base, _TPW)], idx1_v)
        pltpu.sync_copy(flat_hbm.at[pl.ds(base, _TPW)], rows_v)
        c0 = pltpu.async_copy(rows_v, out_hbm.at[idx0_v], sem)
        c1 = pltpu.async_copy(rows_v, out_hbm.at[idx1_v], sem)
        c0.wait()
        c1.wait()

    return k(flat, pos0, pos1)


def _gather_sc(y, pos0, pos1):
    mesh = plsc.VectorSubcoreMesh(core_axis_name="c", subcore_axis_name="s")

    half = _TPW // 2

    @functools.partial(
        pl.kernel, mesh=mesh,
        out_type=[jax.ShapeDtypeStruct((_T, _D), jnp.float32),
                  jax.ShapeDtypeStruct((_T, _D), jnp.float32)],
        scratch_types=[
            pltpu.VMEM((_TPW,), jnp.int32),
            pltpu.VMEM((_TPW,), jnp.int32),
            pltpu.VMEM((half, _D), jnp.float32),
            pltpu.VMEM((half, _D), jnp.float32),
            pltpu.SemaphoreType.DMA,
            pltpu.SemaphoreType.DMA,
        ],
    )
    def k(y_hbm, p0_hbm, p1_hbm, y0_hbm, y1_hbm, idx0_v, idx1_v, r0_v, r1_v,
          semg, sems):
        wid = lax.axis_index("s") * _NC + lax.axis_index("c")
        base = wid * _TPW
        pltpu.sync_copy(p0_hbm.at[pl.ds(base, _TPW)], idx0_v)
        pltpu.sync_copy(p1_hbm.at[pl.ds(base, _TPW)], idx1_v)
        for h in range(2):
            g0 = pltpu.async_copy(
                y_hbm.at[idx0_v.at[pl.ds(h * half, half)]], r0_v, semg)
            g1 = pltpu.async_copy(
                y_hbm.at[idx1_v.at[pl.ds(h * half, half)]], r1_v, semg)
            g0.wait()
            s0 = pltpu.async_copy(
                r0_v, y0_hbm.at[pl.ds(base + h * half, half)], sems)
            g1.wait()
            s1 = pltpu.async_copy(
                r1_v, y1_hbm.at[pl.ds(base + h * half, half)], sems)
            s0.wait()
            s1.wait()

    return k(y, pos0, pos1)


def _combine_body(sh_ref, y0_ref, y1_ref, w01_ref, out_ref):
    wt = jnp.transpose(w01_ref[...])
    w0 = wt[:, 0:1]
    w1 = wt[:, 1:2]
    out_ref[...] = sh_ref[...] + w0 * y0_ref[...] + w1 * y1_ref[...]


def _combine_tc(shared, y0, y1, w01, interpret=False):
    nblk = 4
    rows = _T // nblk
    return pl.pallas_call(
        _combine_body,
        grid=(nblk,),
        in_specs=[
            pl.BlockSpec((rows, _D), lambda i: (i, 0)),
            pl.BlockSpec((rows, _D), lambda i: (i, 0)),
            pl.BlockSpec((rows, _D), lambda i: (i, 0)),
            pl.BlockSpec((2, rows), lambda i: (0, i)),
        ],
        out_specs=pl.BlockSpec((rows, _D), lambda i: (i, 0)),
        out_shape=jax.ShapeDtypeStruct((_T, _D), jnp.float32),
        interpret=interpret,
    )(shared, y0, y1, w01)


@jax.jit
def kernel(hidden_states, router_w, w_gate, w_up, w_down,
           sh_gate, sh_up, sh_down):
    flat = hidden_states.reshape(_T, _D)
    logits, w01, pos0, pos1, te = _router_tc(flat, router_w)
    tile_expert = te.reshape(_NT + 1)
    x_sorted = _dispatch_sc(flat, pos0, pos1)
    shared = _shared_tc(flat, sh_gate, sh_up, sh_down)
    y = _grouped_tc(tile_expert, x_sorted, w_gate, w_up, w_down)
    y0, y1 = _gather_sc(y, pos0, pos1)
    out = _combine_tc(shared, y0, y1, w01)
    return out.reshape(_B, _S, _D), logits


# speedup vs baseline: 2.4117x; 1.9905x over previous
import jax
import jax.numpy as jnp
from jax import lax
from jax.experimental import pallas as pl
from jax.experimental.pallas import tpu as pltpu

_B, _S, _D = 1, 2048, 1024
_E = 8
_FF, _FF_SH = 256, 512
_T = _B * _S


def _silu(x):
    return x * (1.0 / (1.0 + jnp.exp(-x)))


def _moe_body(flat_ref, router_w_ref, wg_ref, wu_ref, wd_ref,
              shg_ref, shu_ref, shd_ref,
              out_ref, logits_ref, comb_ref, flatb_ref):
    e = pl.program_id(0)

    @pl.when(e == 0)
    def _prologue():
        flat = flat_ref[...]
        flatb_ref[...] = flat.astype(jnp.bfloat16)
        logits = jnp.dot(flat, router_w_ref[...],
                         preferred_element_type=jnp.float32)
        logits_ref[...] = logits
        lt = jnp.transpose(logits)
        lmax = jnp.max(lt, axis=0, keepdims=True)
        p = jnp.exp(lt - lmax)
        sub = lax.broadcasted_iota(jnp.int32, (_E, _T), 0)
        m1 = jnp.max(p, axis=0, keepdims=True)
        i1 = jnp.min(jnp.where(p == m1, sub, _E), axis=0, keepdims=True)
        p2 = jnp.where(sub == i1, -jnp.inf, p)
        m2 = jnp.max(p2, axis=0, keepdims=True)
        i2 = jnp.min(jnp.where(p2 == m2, sub, _E), axis=0, keepdims=True)
        s = m1 + m2
        comb_ref[...] = jnp.where(sub == i1, m1 / s, 0.0) + jnp.where(
            sub == i2, m2 / s, 0.0)
        fb = flatb_ref[...]
        g = jnp.dot(fb, shg_ref[...].astype(jnp.bfloat16),
                    preferred_element_type=jnp.float32)
        u = jnp.dot(fb, shu_ref[...].astype(jnp.bfloat16),
                    preferred_element_type=jnp.float32)
        out_ref[...] = jnp.dot((_silu(g) * u).astype(jnp.bfloat16),
                               shd_ref[...].astype(jnp.bfloat16),
                               preferred_element_type=jnp.float32)

    fb = flatb_ref[...]
    col = jnp.transpose(jnp.sum(jnp.where(
        lax.broadcasted_iota(jnp.int32, (_E, _T), 0) == e, comb_ref[...], 0.0),
        axis=0, keepdims=True))
    g = jnp.dot(fb, wg_ref[0].astype(jnp.bfloat16),
                preferred_element_type=jnp.float32)
    u = jnp.dot(fb, wu_ref[0].astype(jnp.bfloat16),
                preferred_element_type=jnp.float32)
    out_ref[...] += col * jnp.dot((_silu(g) * u).astype(jnp.bfloat16),
                                  wd_ref[0].astype(jnp.bfloat16),
                                  preferred_element_type=jnp.float32)


@jax.jit
def kernel(hidden_states, router_w, w_gate, w_up, w_down,
           sh_gate, sh_up, sh_down):
    flat = hidden_states.reshape(_T, _D)
    out, logits = pl.pallas_call(
        _moe_body,
        grid=(_E,),
        in_specs=[
            pl.BlockSpec((_T, _D), lambda e: (0, 0)),
            pl.BlockSpec((_D, _E), lambda e: (0, 0)),
            pl.BlockSpec((1, _D, _FF), lambda e: (e, 0, 0)),
            pl.BlockSpec((1, _D, _FF), lambda e: (e, 0, 0)),
            pl.BlockSpec((1, _FF, _D), lambda e: (e, 0, 0)),
            pl.BlockSpec((_D, _FF_SH), lambda e: (0, 0)),
            pl.BlockSpec((_D, _FF_SH), lambda e: (0, 0)),
            pl.BlockSpec((_FF_SH, _D), lambda e: (0, 0)),
        ],
        out_specs=[
            pl.BlockSpec((_T, _D), lambda e: (0, 0)),
            pl.BlockSpec((_T, _E), lambda e: (0, 0)),
        ],
        out_shape=[
            jax.ShapeDtypeStruct((_T, _D), jnp.float32),
            jax.ShapeDtypeStruct((_T, _E), jnp.float32),
        ],
        scratch_shapes=[pltpu.VMEM((_E, _T), jnp.float32),
                        pltpu.VMEM((_T, _D), jnp.bfloat16)],
        compiler_params=pltpu.CompilerParams(
            dimension_semantics=("arbitrary",),
        ),
    )(flat, router_w, w_gate, w_up, w_down, sh_gate, sh_up, sh_down)
    return out.reshape(_B, _S, _D), logits
